# baseline (speedup 1.0000x reference)
import functools

import jax
import jax.numpy as jnp
from jax import lax
from jax.experimental import pallas as pl
from jax.experimental.pallas import tpu as pltpu
from jax.experimental.pallas import tpu_sc as plsc

B, T_H, T_Q, V, D = 4096, 200, 20, 100000, 64
ZPAD = 256
TT_H = T_H // 8
TT_Q = 3

_info = plsc.get_sparse_core_info()
NC, NS, L = _info.num_cores, _info.num_subcores, _info.num_lanes
NW = NC * NS
ROWS_W = B // NW
NJ = ROWS_W // L


def _splat(x):
  return lax.broadcast_in_dim(jnp.int32(x), (L,), ())


def _bsplat(x):
  return lax.broadcast_in_dim(x, (L,), ())


def _sc_body(th4, qt4, hlen, qlen, table,
             xh, xq, mh4, mq4,
             tokbuf, idxbuf, maskbuf, rows,
             hlen_buf, qlen_buf, semg, semw):
  w = lax.axis_index("s") * NC + lax.axis_index("c")
  pltpu.sync_copy(hlen.at[pl.ds(w * ROWS_W, ROWS_W)], hlen_buf)
  pltpu.sync_copy(qlen.at[pl.ds(w * ROWS_W, ROWS_W)], qlen_buf)

  iota = lax.iota(jnp.int32, L)
  vzero = _splat(0)
  vv = _splat(V)
  vzm = _splat(ZPAD - 1)

  def drain_writes(n):
    for _ in range(n):
      pltpu.make_async_copy(
          rows.at[pl.ds(0, 64), :],
          xh.at[0, pl.ds(0, 64), pl.ds(0, D)], semw).wait()

  def chunk(tt, tok4, len_buf, out3, mask4, guard_drain):
    pltpu.sync_copy(tok4.at[tt, w], tokbuf)
    for ti in range(8):
      t = tt * 8 + ti
      tsp = _bsplat(t)
      for j in range(NJ):
        tok_v = tokbuf[ti, pl.ds(j * L, L)]
        len_v = len_buf[pl.ds(j * L, L)]
        m = tsp < len_v
        keep = jnp.logical_and(m, tok_v != vzero)
        zidx = vv + ((iota + _bsplat(t * 37 + j * L)) & vzm)
        idxbuf[ti, pl.ds(j * L, L)] = jnp.where(keep, tok_v, zidx)
        maskbuf[ti, pl.ds(j * L, L)] = m.astype(jnp.int32)
    pltpu.sync_copy(maskbuf, mask4.at[tt, w])

    if guard_drain:
      @pl.when(tt > 0)
      def _():
        drain_writes(16)
    else:
      drain_writes(16)

    descs = []
    for ti in range(8):
      descs.append(pltpu.async_copy(
          table.at[idxbuf.at[ti, :]],
          rows.at[pl.ds(ti * ROWS_W, ROWS_W), :], semg))
    for d in descs:
      d.wait()
    for ti in range(8):
      t = tt * 8 + ti
      for hh in (0, 1):
        pltpu.async_copy(
            rows.at[pl.ds(ti * ROWS_W + hh * 64, 64), :],
            out3.at[t, pl.ds(w * 64, 64), pl.ds(hh * D, D)], semw)

  @pl.loop(0, TT_H)
  def _hist(g):
    chunk(g, th4, hlen_buf, xh, mh4, True)

  @pl.loop(0, TT_Q)
  def _ques(g):
    chunk(g, qt4, qlen_buf, xq, mq4, False)

  drain_writes(16)


def _tpose_body(x_ref, o_ref, tb):
  x = x_ref[...]
  y = jnp.transpose(x, (0, 2, 1))
  y = y.reshape(tb, 2, D, D)
  y = jnp.transpose(y, (0, 2, 1, 3))
  o_ref[...] = y.reshape(tb, 8, 1, 8, 128)


def _tc_transpose(x2, t, tb):
  return pl.pallas_call(
      functools.partial(_tpose_body, tb=tb),
      grid=(t // tb, NW),
      in_specs=[pl.BlockSpec((tb, D, 128), lambda i, j: (i, j, 0))],
      out_specs=pl.BlockSpec((tb, 8, 1, 8, 128), lambda i, j: (i, 0, j, 0, 0)),
      out_shape=jax.ShapeDtypeStruct((t, 8, NW, 8, 128), jnp.float32),
  )(x2)


@jax.jit
def _encode(ques_tokens, hist_tokens, ques_len, hist_len, table):
  th4 = hist_tokens.reshape(32, 128, TT_H, 8).transpose(2, 0, 3, 1)
  qt4 = jnp.pad(ques_tokens, ((0, 0), (0, 4))).reshape(
      32, 128, TT_Q, 8).transpose(2, 0, 3, 1)
  table_ext = jnp.concatenate(
      [table, jnp.zeros((ZPAD, D), jnp.float32)], axis=0)

  mesh = plsc.VectorSubcoreMesh(core_axis_name="c", subcore_axis_name="s")
  kfn = pl.kernel(
      _sc_body,
      out_type=[
          jax.ShapeDtypeStruct((T_H, B // 2, 2 * D), jnp.float32),
          jax.ShapeDtypeStruct((TT_Q * 8, B // 2, 2 * D), jnp.float32),
          jax.ShapeDtypeStruct((TT_H, NW, 8, 128), jnp.int32),
          jax.ShapeDtypeStruct((TT_Q, NW, 8, 128), jnp.int32),
      ],
      mesh=mesh,
      compiler_params=pltpu.CompilerParams(
          use_tc_tiling_on_sc=False, needs_layout_passes=False),
      scratch_types=[
          pltpu.VMEM((8, 128), jnp.int32),
          pltpu.VMEM((8, 128), jnp.int32),
          pltpu.VMEM((8, 128), jnp.int32),
          pltpu.VMEM((8 * ROWS_W, D), jnp.float32),
          pltpu.VMEM((ROWS_W,), jnp.int32),
          pltpu.VMEM((ROWS_W,), jnp.int32),
          pltpu.SemaphoreType.DMA,
          pltpu.SemaphoreType.DMA,
      ],
  )
  xh, xq, mh4, mq4 = kfn(th4, qt4, hist_len, ques_len, table_ext)

  oh5 = _tc_transpose(xh, T_H, 8)
  oq5 = _tc_transpose(xq, T_Q, 4)

  hist = oh5.transpose(2, 4, 0, 1, 3).reshape(B, T_H, D)
  ques = oq5.transpose(2, 4, 0, 1, 3).reshape(B, T_Q, D)
  hist_mask = mh4.transpose(1, 3, 0, 2).reshape(B, T_H)
  ques_mask = mq4.transpose(1, 3, 0, 2).reshape(B, 24)[:, :T_Q]
  return (hist, ques, hist_mask, ques_mask)


def kernel(ques_tokens, hist_tokens, ques_len, hist_len, text_embedding_weight):
  ques_tokens = ques_tokens.astype(jnp.int32)
  hist_tokens = hist_tokens.astype(jnp.int32)
  ques_len = ques_len.astype(jnp.int32)
  hist_len = hist_len.astype(jnp.int32)
  return _encode(ques_tokens, hist_tokens, ques_len, hist_len,
                 text_embedding_weight)


# speedup vs baseline: 1.2798x; 1.2798x over previous
import jax
import jax.numpy as jnp
from jax import lax
from jax.experimental import pallas as pl
from jax.experimental.pallas import tpu as pltpu
from jax.experimental.pallas import tpu_sc as plsc

B, T_H, T_Q, V, D = 4096, 200, 20, 100000, 64

_info = plsc.get_sparse_core_info()
NC, NS, L = _info.num_cores, _info.num_subcores, _info.num_lanes
NW = NC * NS
ROWS_W = B // NW
CH = 512
GSUB = 128
TOK_H = ROWS_W * T_H
TOK_Q = ROWS_W * T_Q


def _process_stream(wid, T, n_tok_w, tok_hbm, table, out_hbm, mask_hbm,
                    len_buf, tok_buf, rows_buf, mask_buf, keep_buf, sem):
  n_chunks = n_tok_w // CH
  base_w = wid * n_tok_w

  @pl.loop(0, n_chunks)
  def _chunk(g):
    tok_base = base_w + g * CH
    pltpu.sync_copy(tok_hbm.at[pl.ds(tok_base, CH)], tok_buf)

    descs = []
    for j in range(CH // GSUB):
      descs.append(pltpu.async_copy(
          table.at[tok_buf.at[pl.ds(j * GSUB, GSUB)]],
          rows_buf.at[pl.ds(j * GSUB, GSUB)], sem))

    base_v = lax.broadcast_in_dim(g * CH, (L,), ())
    t_v = jnp.full((L,), T, jnp.int32)
    zero_v = jnp.zeros((L,), jnp.int32)
    for i in range(CH // L):
      offs = jnp.arange(i * L, (i + 1) * L, dtype=jnp.int32)
      pos = base_v + offs
      r = lax.div(pos, t_v)
      t = pos - r * t_v
      len_v = plsc.load_gather(len_buf, [r])
      tok_v = tok_buf[pl.ds(i * L, L)]
      m = t < len_v
      mask_buf[pl.ds(i * L, L)] = m.astype(jnp.int32)
      keep = jnp.logical_and(m, tok_v != zero_v)
      keep_buf[pl.ds(i * L, L)] = keep.astype(jnp.float32)

    for d in descs:
      d.wait()

    @pl.loop(0, CH // L)
    def _mul(kb):
      kv = keep_buf[pl.ds(kb * L, L)]
      base = kb * L
      for lane in range(L):
        ksv = lax.broadcast_in_dim(kv[lane], (L,), ())
        for q in range(D // L):
          row = base + lane
          rows_buf[row, pl.ds(q * L, L)] = rows_buf[row, pl.ds(q * L, L)] * ksv

    pltpu.sync_copy(rows_buf, out_hbm.at[pl.ds(tok_base, CH)])
    pltpu.sync_copy(mask_buf, mask_hbm.at[pl.ds(tok_base, CH)])


def _sc_body(tok_h, tok_q, hlen, qlen, table,
             out_h, out_q, mask_h, mask_q,
             tok_buf, rows_buf, mask_buf, keep_buf, hlen_buf, qlen_buf, sem):
  wid = lax.axis_index("s") * NC + lax.axis_index("c")
  pltpu.sync_copy(hlen.at[pl.ds(wid * ROWS_W, ROWS_W)], hlen_buf)
  pltpu.sync_copy(qlen.at[pl.ds(wid * ROWS_W, ROWS_W)], qlen_buf)

  _process_stream(wid, T_H, TOK_H, tok_h, table, out_h, mask_h,
                  hlen_buf, tok_buf, rows_buf, mask_buf, keep_buf, sem)
  _process_stream(wid, T_Q, TOK_Q, tok_q, table, out_q, mask_q,
                  qlen_buf, tok_buf, rows_buf, mask_buf, keep_buf, sem)


@jax.jit
def _encode(ques_tokens, hist_tokens, ques_len, hist_len, table):
  mesh = plsc.VectorSubcoreMesh(core_axis_name="c", subcore_axis_name="s")
  kfn = pl.kernel(
      _sc_body,
      out_type=[
          jax.ShapeDtypeStruct((B * T_H, D), jnp.float32),
          jax.ShapeDtypeStruct((B * T_Q, D), jnp.float32),
          jax.ShapeDtypeStruct((B * T_H,), jnp.int32),
          jax.ShapeDtypeStruct((B * T_Q,), jnp.int32),
      ],
      mesh=mesh,
      compiler_params=pltpu.CompilerParams(
          use_tc_tiling_on_sc=False, needs_layout_passes=False),
      scratch_types=[
          pltpu.VMEM((CH,), jnp.int32),
          pltpu.VMEM((CH, D), jnp.float32),
          pltpu.VMEM((CH,), jnp.int32),
          pltpu.VMEM((CH,), jnp.float32),
          pltpu.VMEM((ROWS_W,), jnp.int32),
          pltpu.VMEM((ROWS_W,), jnp.int32),
          pltpu.SemaphoreType.DMA,
      ],
  )
  out_h, out_q, mask_h, mask_q = kfn(
      hist_tokens.reshape(-1), ques_tokens.reshape(-1),
      hist_len, ques_len, table)
  return (out_h.reshape(B, T_H, D), out_q.reshape(B, T_Q, D),
          mask_h.reshape(B, T_H), mask_q.reshape(B, T_Q))


def kernel(ques_tokens, hist_tokens, ques_len, hist_len, text_embedding_weight):
  ques_tokens = ques_tokens.astype(jnp.int32)
  hist_tokens = hist_tokens.astype(jnp.int32)
  ques_len = ques_len.astype(jnp.int32)
  hist_len = hist_len.astype(jnp.int32)
  return _encode(ques_tokens, hist_tokens, ques_len, hist_len,
                 text_embedding_weight)


# speedup vs baseline: 1.3505x; 1.0552x over previous
import jax
import jax.numpy as jnp
from jax import lax
from jax.experimental import pallas as pl
from jax.experimental.pallas import tpu as pltpu
from jax.experimental.pallas import tpu_sc as plsc

B, T_H, T_Q, V, D = 4096, 200, 20, 100000, 64

_info = plsc.get_sparse_core_info()
NC, NS, L = _info.num_cores, _info.num_subcores, _info.num_lanes
NW = NC * NS
ROWS_W = B // NW
CH = 512
GSUB = 128
TOK_H = ROWS_W * T_H
TOK_Q = ROWS_W * T_Q


def _chunk_body(g, base_w, T, tok_hbm, table, out_hbm, mask_hbm, len_buf,
                tok_buf, rows_buf, mask_buf, keep_buf, sem,
                out_sem=None):
  tok_base = base_w + g * CH
  pltpu.sync_copy(tok_hbm.at[pl.ds(tok_base, CH)], tok_buf)

  descs = []
  for j in range(CH // GSUB):
    descs.append(pltpu.async_copy(
        table.at[tok_buf.at[pl.ds(j * GSUB, GSUB)]],
        rows_buf.at[pl.ds(j * GSUB, GSUB)], sem))

  base_v = lax.broadcast_in_dim(g * CH, (L,), ())
  t_v = jnp.full((L,), T, jnp.int32)
  zero_v = jnp.zeros((L,), jnp.int32)
  for i in range(CH // L):
    offs = jnp.arange(i * L, (i + 1) * L, dtype=jnp.int32)
    pos = base_v + offs
    r = lax.div(pos, t_v)
    t = pos - r * t_v
    len_v = plsc.load_gather(len_buf, [r])
    tok_v = tok_buf[pl.ds(i * L, L)]
    m = t < len_v
    mask_buf[pl.ds(i * L, L)] = m.astype(jnp.int32)
    keep = jnp.logical_and(m, tok_v != zero_v)
    keep_buf[pl.ds(i * L, L)] = keep.astype(jnp.float32)

  for d in descs:
    d.wait()

  @pl.loop(0, CH // L)
  def _mul(kb):
    kv = keep_buf[pl.ds(kb * L, L)]
    base = kb * L
    for lane in range(L):
      ksv = lax.broadcast_in_dim(kv[lane], (L,), ())
      for q in range(D // L):
        row = base + lane
        rows_buf[row, pl.ds(q * L, L)] = rows_buf[row, pl.ds(q * L, L)] * ksv

  if out_sem is None:
    pltpu.sync_copy(rows_buf, out_hbm.at[pl.ds(tok_base, CH)])
    pltpu.sync_copy(mask_buf, mask_hbm.at[pl.ds(tok_base, CH)])
  else:
    pltpu.async_copy(rows_buf, out_hbm.at[pl.ds(tok_base, CH)], out_sem)
    pltpu.async_copy(mask_buf, mask_hbm.at[pl.ds(tok_base, CH)], out_sem)


def _sc_body(tok_h, tok_q, hlen, qlen, table,
             out_h, out_q, mask_h, mask_q,
             tok_a, rows_a, mask_a, keep_a,
             tok_b, rows_b, mask_b, keep_b,
             hlen_buf, qlen_buf, semg, semw_a, semw_b):
  wid = lax.axis_index("s") * NC + lax.axis_index("c")
  pltpu.sync_copy(hlen.at[pl.ds(wid * ROWS_W, ROWS_W)], hlen_buf)
  pltpu.sync_copy(qlen.at[pl.ds(wid * ROWS_W, ROWS_W)], qlen_buf)

  bufsets = (
      (tok_a, rows_a, mask_a, keep_a, semw_a),
      (tok_b, rows_b, mask_b, keep_b, semw_b),
  )

  def drain(rows_buf, mask_buf, semw):
    pltpu.make_async_copy(rows_buf, out_h.at[pl.ds(0, CH)], semw).wait()
    pltpu.make_async_copy(mask_buf, mask_h.at[pl.ds(0, CH)], semw).wait()

  base_h = wid * TOK_H

  @pl.loop(0, TOK_H // CH // 2)
  def _hist(gg):
    for p, (tkb, rwb, mkb, kpb, semw) in enumerate(bufsets):
      @pl.when(gg > 0)
      def _():
        drain(rwb, mkb, semw)
      _chunk_body(2 * gg + p, base_h, T_H, tok_h, table, out_h, mask_h,
                  hlen_buf, tkb, rwb, mkb, kpb, semg, out_sem=semw)

  for _, rwb, mkb, _unused, semw in (
      (0, rows_a, mask_a, 0, semw_a), (0, rows_b, mask_b, 0, semw_b)):
    drain(rwb, mkb, semw)

  base_q = wid * TOK_Q

  @pl.loop(0, TOK_Q // CH)
  def _ques(g):
    _chunk_body(g, base_q, T_Q, tok_q, table, out_q, mask_q,
                qlen_buf, tok_a, rows_a, mask_a, keep_a, semg)


@jax.jit
def _encode(ques_tokens, hist_tokens, ques_len, hist_len, table):
  mesh = plsc.VectorSubcoreMesh(core_axis_name="c", subcore_axis_name="s")
  kfn = pl.kernel(
      _sc_body,
      out_type=[
          jax.ShapeDtypeStruct((B * T_H, D), jnp.float32),
          jax.ShapeDtypeStruct((B * T_Q, D), jnp.float32),
          jax.ShapeDtypeStruct((B * T_H,), jnp.int32),
          jax.ShapeDtypeStruct((B * T_Q,), jnp.int32),
      ],
      mesh=mesh,
      compiler_params=pltpu.CompilerParams(
          use_tc_tiling_on_sc=False, needs_layout_passes=False),
      scratch_types=[
          pltpu.VMEM((CH,), jnp.int32),
          pltpu.VMEM((CH, D), jnp.float32),
          pltpu.VMEM((CH,), jnp.int32),
          pltpu.VMEM((CH,), jnp.float32),
          pltpu.VMEM((CH,), jnp.int32),
          pltpu.VMEM((CH, D), jnp.float32),
          pltpu.VMEM((CH,), jnp.int32),
          pltpu.VMEM((CH,), jnp.float32),
          pltpu.VMEM((ROWS_W,), jnp.int32),
          pltpu.VMEM((ROWS_W,), jnp.int32),
          pltpu.SemaphoreType.DMA,
          pltpu.SemaphoreType.DMA,
          pltpu.SemaphoreType.DMA,
      ],
  )
  out_h, out_q, mask_h, mask_q = kfn(
      hist_tokens.reshape(-1), ques_tokens.reshape(-1),
      hist_len, ques_len, table)
  return (out_h.reshape(B, T_H, D), out_q.reshape(B, T_Q, D),
          mask_h.reshape(B, T_H), mask_q.reshape(B, T_Q))


def kernel(ques_tokens, hist_tokens, ques_len, hist_len, text_embedding_weight):
  ques_tokens = ques_tokens.astype(jnp.int32)
  hist_tokens = hist_tokens.astype(jnp.int32)
  ques_len = ques_len.astype(jnp.int32)
  hist_len = hist_len.astype(jnp.int32)
  return _encode(ques_tokens, hist_tokens, ques_len, hist_len,
                 text_embedding_weight)
